# Initial kernel scaffold; baseline (speedup 1.0000x reference)
import functools

import jax
import jax.numpy as jnp
from jax import lax
from jax.experimental import pallas as pl
from jax.experimental.pallas import tpu as pltpu
from jax.experimental.pallas import tpu_sc as plsc

NUM_TABLE_ROWS = 1000000
DIM = 64
BATCH = 16384 * 50

_info = plsc.get_sparse_core_info()
NUM_CORES = _info.num_cores
NUM_SUBCORES = _info.num_subcores
NUM_WORKERS = NUM_CORES * NUM_SUBCORES

ROWS_PER_WORKER = BATCH // NUM_WORKERS
CHUNK = 1024
NUM_CHUNKS = ROWS_PER_WORKER // CHUNK


@functools.partial(
    pl.kernel,
    out_type=jax.ShapeDtypeStruct((BATCH, DIM), jnp.float32),
    mesh=plsc.VectorSubcoreMesh(core_axis_name="c", subcore_axis_name="s"),
    scratch_types=[
        pltpu.VMEM((CHUNK,), jnp.int32),
        pltpu.VMEM((CHUNK, DIM), jnp.float32),
        pltpu.SemaphoreType.DMA,
    ],
)
def _gather_kernel(table_hbm, idx_hbm, out_hbm, idx_v, rows_v, sem):
    wid = lax.axis_index("s") * NUM_CORES + lax.axis_index("c")
    base = wid * ROWS_PER_WORKER

    def chunk_body(g, carry):
        off = base + g * CHUNK
        pltpu.sync_copy(idx_hbm.at[pl.ds(off, CHUNK)], idx_v)
        pltpu.async_copy(table_hbm.at[idx_v], rows_v, sem).wait()
        pltpu.sync_copy(rows_v, out_hbm.at[pl.ds(off, CHUNK)])
        return carry

    lax.fori_loop(0, NUM_CHUNKS, chunk_body, 0)


def kernel(token_ids, weights):
    idx = token_ids.reshape(-1).astype(jnp.int32)
    out = _gather_kernel(weights, idx)
    return out.reshape(token_ids.shape + (DIM,))


# speedup vs baseline: 1.8434x; 1.8434x over previous
import functools

import jax
import jax.numpy as jnp
from jax import lax
from jax.experimental import pallas as pl
from jax.experimental.pallas import tpu as pltpu
from jax.experimental.pallas import tpu_sc as plsc

NUM_TABLE_ROWS = 1000000
DIM = 64
BATCH = 16384 * 50

_info = plsc.get_sparse_core_info()
NUM_CORES = _info.num_cores
NUM_SUBCORES = _info.num_subcores
NUM_WORKERS = NUM_CORES * NUM_SUBCORES

ROWS_PER_WORKER = BATCH // NUM_WORKERS
CHUNK = 1024
NUM_CHUNKS = ROWS_PER_WORKER // CHUNK


@functools.partial(
    pl.kernel,
    out_type=jax.ShapeDtypeStruct((BATCH, DIM), jnp.float32),
    mesh=plsc.VectorSubcoreMesh(core_axis_name="c", subcore_axis_name="s"),
    compiler_params=pltpu.CompilerParams(use_tc_tiling_on_sc=False),
    scratch_types=[
        pltpu.VMEM((CHUNK,), jnp.int32),
        pltpu.VMEM((CHUNK, DIM), jnp.float32),
        pltpu.SemaphoreType.DMA,
    ],
)
def _gather_kernel(table_hbm, idx_hbm, out_hbm, idx_v, rows_v, sem):
    wid = lax.axis_index("s") * NUM_CORES + lax.axis_index("c")
    base = wid * ROWS_PER_WORKER

    def chunk_body(g, carry):
        off = base + g * CHUNK
        pltpu.sync_copy(idx_hbm.at[pl.ds(off, CHUNK)], idx_v)
        pltpu.async_copy(table_hbm.at[idx_v], rows_v, sem).wait()
        pltpu.sync_copy(rows_v, out_hbm.at[pl.ds(off, CHUNK)])
        return carry

    lax.fori_loop(0, NUM_CHUNKS, chunk_body, 0)


def kernel(token_ids, weights):
    idx = token_ids.reshape(-1).astype(jnp.int32)
    out = _gather_kernel(weights, idx)
    return out.reshape(token_ids.shape + (DIM,))


# speedup vs baseline: 1.8736x; 1.0164x over previous
import functools

import jax
import jax.numpy as jnp
from jax import lax
from jax.experimental import pallas as pl
from jax.experimental.pallas import tpu as pltpu
from jax.experimental.pallas import tpu_sc as plsc

DIM = 64
BATCH = 16384 * 50

_info = plsc.get_sparse_core_info()
NUM_CORES = _info.num_cores
NUM_SUBCORES = _info.num_subcores
NUM_WORKERS = NUM_CORES * NUM_SUBCORES

ROWS_PER_WORKER = BATCH // NUM_WORKERS
CHUNK = 400
NUM_CHUNKS = ROWS_PER_WORKER // CHUNK
NBUF = 4
SHIFT = 2
NUM_ROUNDS = NUM_CHUNKS // NBUF


@functools.partial(
    pl.kernel,
    out_type=jax.ShapeDtypeStruct((BATCH, DIM), jnp.float32),
    mesh=plsc.VectorSubcoreMesh(core_axis_name="c", subcore_axis_name="s"),
    compiler_params=pltpu.CompilerParams(use_tc_tiling_on_sc=False),
    scratch_types=(
        [pltpu.VMEM((ROWS_PER_WORKER,), jnp.int32)]
        + [pltpu.VMEM((CHUNK, DIM), jnp.float32) for _ in range(NBUF)]
        + [pltpu.SemaphoreType.DMA for _ in range(2 * NBUF)]
    ),
)
def _gather_kernel(table_hbm, idx_hbm, out_hbm, idx_all, *bufs_and_sems):
    rows = bufs_and_sems[:NBUF]
    gsem = bufs_and_sems[NBUF:2 * NBUF]
    wsem = bufs_and_sems[2 * NBUF:]

    wid = lax.axis_index("s") * NUM_CORES + lax.axis_index("c")
    base = wid * ROWS_PER_WORKER
    pltpu.sync_copy(idx_hbm.at[pl.ds(base, ROWS_PER_WORKER)], idx_all)

    def start_gather(g, b):
        pltpu.async_copy(
            table_hbm.at[idx_all.at[pl.ds(g * CHUNK, CHUNK)]], rows[b], gsem[b])

    def wait_gather(g, b):
        pltpu.make_async_copy(
            table_hbm.at[idx_all.at[pl.ds(g * CHUNK, CHUNK)]], rows[b], gsem[b]).wait()

    def start_writeback(g, b):
        pltpu.async_copy(rows[b], out_hbm.at[pl.ds(base + g * CHUNK, CHUNK)], wsem[b])

    def wait_writeback(g, b):
        pltpu.make_async_copy(
            rows[b], out_hbm.at[pl.ds(base + g * CHUNK, CHUNK)], wsem[b]).wait()

    for b in range(NBUF):
        start_gather(b, b)
        if b >= SHIFT:
            wait_gather(b - SHIFT, b - SHIFT)
            start_writeback(b - SHIFT, b - SHIFT)

    def round_body(i, carry):
        g0 = i * NBUF
        for b in range(NBUF):
            g = g0 + b
            wait_writeback(g - NBUF, b)
            start_gather(g, b)
            bp = (b - SHIFT) % NBUF
            wait_gather(g - SHIFT, bp)
            start_writeback(g - SHIFT, bp)
        return carry

    lax.fori_loop(1, NUM_ROUNDS, round_body, 0)

    for j in range(SHIFT):
        g = NUM_CHUNKS - SHIFT + j
        b = g % NBUF
        wait_gather(g, b)
        start_writeback(g, b)
    for b in range(NBUF):
        g = NUM_CHUNKS - NBUF + b
        wait_writeback(g, b)


def kernel(token_ids, weights):
    idx = token_ids.reshape(-1).astype(jnp.int32)
    out = _gather_kernel(weights, idx)
    return out.reshape(token_ids.shape + (DIM,))
